# Initial kernel scaffold; baseline (speedup 1.0000x reference)
import jax
import jax.numpy as jnp
from jax.experimental import pallas as pl


def kernel(inputs, table):
    raise NotImplementedError("write your pallas kernel here")


# speedup vs baseline: 9.0329x; 9.0329x over previous
import functools

import jax
import jax.numpy as jnp
from jax import lax
from jax.experimental import pallas as pl
from jax.experimental.pallas import tpu as pltpu
from jax.experimental.pallas import tpu_sc as plsc

N = 819200
DIM = 32
NUM_CORES = 2
NUM_SUBCORES = 16
NW = NUM_CORES * NUM_SUBCORES
B_PER_W = N // NW
CHUNK = 3200
NCHUNK = B_PER_W // CHUNK


def _sc_gather(inputs, table):
    mesh = plsc.VectorSubcoreMesh(core_axis_name="c", subcore_axis_name="s")

    @functools.partial(
        pl.kernel,
        mesh=mesh,
        out_type=jax.ShapeDtypeStruct((N, DIM), jnp.float32),
        scratch_types=[
            pltpu.VMEM((CHUNK,), jnp.int32),
            pltpu.VMEM((CHUNK, DIM), jnp.float32),
            pltpu.SemaphoreType.DMA,
        ],
        compiler_params=pltpu.CompilerParams(use_tc_tiling_on_sc=False),
    )
    def k(idx_hbm, table_hbm, out_hbm, idx_v, rows_v, sem):
        wid = lax.axis_index("s") * NUM_CORES + lax.axis_index("c")
        base = wid * B_PER_W
        for i in range(NCHUNK):
            off = base + i * CHUNK
            pltpu.sync_copy(idx_hbm.at[pl.ds(off, CHUNK)], idx_v)
            pltpu.async_copy(table_hbm.at[idx_v], rows_v, sem).wait()
            pltpu.sync_copy(rows_v, out_hbm.at[pl.ds(off, CHUNK)])

    return k(inputs, table)


def kernel(inputs, table):
    return _sc_gather(inputs, table)


# speedup vs baseline: 9.0390x; 1.0007x over previous
import functools

import jax
import jax.numpy as jnp
from jax import lax
from jax.experimental import pallas as pl
from jax.experimental.pallas import tpu as pltpu
from jax.experimental.pallas import tpu_sc as plsc

N = 819200
DIM = 32
NUM_CORES = 2
NUM_SUBCORES = 16
NW = NUM_CORES * NUM_SUBCORES
B_PER_W = N // NW
CHUNK = 1600
NCHUNK = B_PER_W // CHUNK
NBUF = 2


def _sc_gather(inputs, table):
    mesh = plsc.VectorSubcoreMesh(core_axis_name="c", subcore_axis_name="s")

    scratch = (
        [pltpu.VMEM((CHUNK,), jnp.int32) for _ in range(NBUF)]
        + [pltpu.VMEM((CHUNK, DIM), jnp.float32) for _ in range(NBUF)]
        + [pltpu.SemaphoreType.DMA for _ in range(3 * NBUF)]
    )

    @functools.partial(
        pl.kernel,
        mesh=mesh,
        out_type=jax.ShapeDtypeStruct((N, DIM), jnp.float32),
        scratch_types=scratch,
        compiler_params=pltpu.CompilerParams(use_tc_tiling_on_sc=False),
    )
    def k(idx_hbm, table_hbm, out_hbm, *sc):
        idx_bufs = sc[:NBUF]
        row_bufs = sc[NBUF:2 * NBUF]
        isems = sc[2 * NBUF:3 * NBUF]
        gsems = sc[3 * NBUF:4 * NBUF]
        wsems = sc[4 * NBUF:5 * NBUF]
        wid = lax.axis_index("s") * NUM_CORES + lax.axis_index("c")
        base = wid * B_PER_W

        icopy = [None] * NCHUNK
        wb = [None] * NCHUNK
        for i in range(NBUF):
            icopy[i] = pltpu.make_async_copy(
                idx_hbm.at[pl.ds(base + i * CHUNK, CHUNK)], idx_bufs[i], isems[i])
            icopy[i].start()
        for i in range(NCHUNK):
            b = i % NBUF
            if i >= NBUF:
                wb[i - NBUF].wait()
            icopy[i].wait()
            g = pltpu.make_async_copy(
                table_hbm.at[idx_bufs[b]], row_bufs[b], gsems[b])
            g.start()
            g.wait()
            if i + NBUF < NCHUNK:
                icopy[i + NBUF] = pltpu.make_async_copy(
                    idx_hbm.at[pl.ds(base + (i + NBUF) * CHUNK, CHUNK)],
                    idx_bufs[b], isems[b])
                icopy[i + NBUF].start()
            wb[i] = pltpu.make_async_copy(
                row_bufs[b], out_hbm.at[pl.ds(base + i * CHUNK, CHUNK)], wsems[b])
            wb[i].start()
        for i in range(NCHUNK - NBUF, NCHUNK):
            wb[i].wait()

    return k(inputs, table)


def kernel(inputs, table):
    return _sc_gather(inputs, table)
